# Initial kernel scaffold; baseline (speedup 1.0000x reference)
import jax
import jax.numpy as jnp
from jax.experimental import pallas as pl


def kernel(input, feat, level_x, level_y, level_z, time_table, W, b):
    raise NotImplementedError("write your pallas kernel here")


# speedup vs baseline: 1.3844x; 1.3844x over previous
import functools

import jax
import jax.numpy as jnp
from jax import lax
from jax.experimental import pallas as pl
from jax.experimental.pallas import tpu as pltpu
from jax.experimental.pallas import tpu_sc as plsc

_LEVELS = 1024
_TS = 64
_D = 10240
_N = 64
_NW = 32
_SPW = _N // _NW
_LANES = 16
_CHUNKS = _D // _LANES


def _sc_body(ix_hbm, iy_hbm, iz_hbm, it_hbm, lx_hbm, ly_hbm, lz_hbm, tt_hbm,
             out_hbm, idx_v, xr, yr, zr, tr, acc, sem):
    w = lax.axis_index("s") * 2 + lax.axis_index("c")
    pltpu.sync_copy(ix_hbm.at[w], idx_v.at[0])
    pltpu.sync_copy(iy_hbm.at[w], idx_v.at[1])
    pltpu.sync_copy(iz_hbm.at[w], idx_v.at[2])
    pltpu.sync_copy(it_hbm.at[w], idx_v.at[3])
    cx = pltpu.async_copy(lx_hbm.at[idx_v.at[0]], xr, sem)
    cy = pltpu.async_copy(ly_hbm.at[idx_v.at[1]], yr, sem)
    cz = pltpu.async_copy(lz_hbm.at[idx_v.at[2]], zr, sem)
    ct = pltpu.async_copy(tt_hbm.at[idx_v.at[3]], tr, sem)
    cx.wait()
    cy.wait()
    cz.wait()
    ct.wait()

    def step(i, carry):
        s = pl.ds(i * _LANES, _LANES)
        a0 = (xr[0, s] + yr[0, s] + zr[0, s]) * tr[0, s]
        a1 = (xr[1, s] + yr[1, s] + zr[1, s]) * tr[1, s]
        acc[s] = a0 * a1
        return carry

    lax.fori_loop(0, _CHUNKS, step, 0)
    pltpu.sync_copy(acc, out_hbm.at[w])


def _sc_partials(ix, iy, iz, it, lx, ly, lz, tt):
    mesh = plsc.VectorSubcoreMesh(core_axis_name="c", subcore_axis_name="s")
    f = pl.kernel(
        _sc_body,
        out_type=jax.ShapeDtypeStruct((_NW, _D), jnp.float32),
        mesh=mesh,
        scratch_types=[
            pltpu.VMEM((4, _SPW), jnp.int32),
            pltpu.VMEM((_SPW, _D), jnp.float32),
            pltpu.VMEM((_SPW, _D), jnp.float32),
            pltpu.VMEM((_SPW, _D), jnp.float32),
            pltpu.VMEM((_SPW, _D), jnp.float32),
            pltpu.VMEM((_D,), jnp.float32),
            pltpu.SemaphoreType.DMA,
        ],
    )
    return f(ix, iy, iz, it, lx, ly, lz, tt)


def _tc_body(f_ref, w_ref, b_ref, p_ref, o_ref):
    p = jnp.dot(f_ref[...], w_ref[...], preferred_element_type=jnp.float32)
    fbind = jnp.cos(p + b_ref[...]) * jnp.sin(p)
    tot = p_ref[0]
    for i in range(1, _NW):
        tot = tot * p_ref[i]
    o_ref[...] = jnp.where(tot * fbind > 0, 1.0, -1.0).astype(jnp.float32)


def _tc_finish(featp, wt, b2, partials):
    return pl.pallas_call(
        _tc_body,
        out_shape=jax.ShapeDtypeStruct((1, _D), jnp.float32),
        in_specs=[
            pl.BlockSpec(memory_space=pltpu.VMEM),
            pl.BlockSpec(memory_space=pltpu.VMEM),
            pl.BlockSpec(memory_space=pltpu.VMEM),
            pl.BlockSpec(memory_space=pltpu.VMEM),
        ],
        out_specs=pl.BlockSpec(memory_space=pltpu.VMEM),
    )(featp, wt, b2, partials)


def _vti(v, low, high, n):
    return jnp.round(jnp.clip((v - low) / (high - low), 0.0, 1.0) * (n - 1)).astype(jnp.int32)


def kernel(input, feat, level_x, level_y, level_z, time_table, W, b):
    x_signal = jnp.clip(input[:, 1], -5.0, 5.0)
    y_signal = jnp.clip(input[:, 2], -5.0, 5.0)
    z_signal = jnp.clip(input[:, 3], -5.0, 5.0)
    ix = _vti(x_signal, -5.0, 5.0, _LEVELS).reshape(_NW, _SPW)
    iy = _vti(y_signal, -5.0, 5.0, _LEVELS).reshape(_NW, _SPW)
    iz = _vti(z_signal, -5.0, 5.0, _LEVELS).reshape(_NW, _SPW)
    it = _vti(input[:, 0], 0.0, float(_TS), _TS).reshape(_NW, _SPW)

    partials = _sc_partials(ix, iy, iz, it, level_x, level_y, level_z, time_table)

    featp = feat.reshape(1, 6)
    wt = W.T
    b2 = b.reshape(1, _D)
    out = _tc_finish(featp, wt, b2, partials)
    return out.reshape(-1)


# speedup vs baseline: 1.7506x; 1.2645x over previous
import functools

import jax
import jax.numpy as jnp
from jax import lax
from jax.experimental import pallas as pl
from jax.experimental.pallas import tpu as pltpu
from jax.experimental.pallas import tpu_sc as plsc

_LEVELS = 1024
_TS = 64
_D = 10240
_N = 64
_NW = 32
_SPW = _N // _NW
_LANES = 16
_CHUNKS = _D // _LANES
_UNROLL = 4


def _sc_body(idx_hbm, lx_hbm, ly_hbm, lz_hbm, tt_hbm,
             out_hbm, idx_v, x0, y0, z0, t0, x1, y1, z1, t1, acc, sem0, sem1):
    w = lax.axis_index("s") * 2 + lax.axis_index("c")
    pltpu.sync_copy(idx_hbm.at[w], idx_v)
    tabs = (lx_hbm, ly_hbm, lz_hbm, tt_hbm)
    waits0 = []
    waits1 = []
    for k, (tab, r) in enumerate(zip(tabs, (x0, y0, z0, t0))):
        waits0.append(pltpu.async_copy(
            tab.at[idx_v.at[0, pl.ds(k, 1)]], r, sem0))
    for k, (tab, r) in enumerate(zip(tabs, (x1, y1, z1, t1))):
        waits1.append(pltpu.async_copy(
            tab.at[idx_v.at[1, pl.ds(k, 1)]], r, sem1))
    for h in waits0:
        h.wait()

    @plsc.parallel_loop(0, _CHUNKS, 1, unroll=_UNROLL)
    def _loop0(i):
        s = pl.ds(i * _LANES, _LANES)
        acc[s] = (x0[0, s] + y0[0, s] + z0[0, s]) * t0[0, s]

    for h in waits1:
        h.wait()

    @plsc.parallel_loop(0, _CHUNKS, 1, unroll=_UNROLL)
    def _loop1(i):
        s = pl.ds(i * _LANES, _LANES)
        acc[s] = acc[s] * ((x1[0, s] + y1[0, s] + z1[0, s]) * t1[0, s])

    pltpu.sync_copy(acc, out_hbm.at[w])


def _sc_partials(idx, lx, ly, lz, tt):
    mesh = plsc.VectorSubcoreMesh(core_axis_name="c", subcore_axis_name="s")
    f = pl.kernel(
        _sc_body,
        out_type=jax.ShapeDtypeStruct((_NW, _D), jnp.float32),
        mesh=mesh,
        scratch_types=[
            pltpu.VMEM((_SPW, 4), jnp.int32),
            pltpu.VMEM((1, _D), jnp.float32),
            pltpu.VMEM((1, _D), jnp.float32),
            pltpu.VMEM((1, _D), jnp.float32),
            pltpu.VMEM((1, _D), jnp.float32),
            pltpu.VMEM((1, _D), jnp.float32),
            pltpu.VMEM((1, _D), jnp.float32),
            pltpu.VMEM((1, _D), jnp.float32),
            pltpu.VMEM((1, _D), jnp.float32),
            pltpu.VMEM((_D,), jnp.float32),
            pltpu.SemaphoreType.DMA,
            pltpu.SemaphoreType.DMA,
        ],
    )
    return f(idx, lx, ly, lz, tt)


def _fb_body(f_ref, w_ref, b_ref, o_ref):
    p = jnp.dot(f_ref[...], w_ref[...], preferred_element_type=jnp.float32)
    o_ref[...] = jnp.cos(p + b_ref[...]) * jnp.sin(p)


def _comb_body(p_ref, fb_ref, o_ref):
    tot = p_ref[0]
    for i in range(1, _NW):
        tot = tot * p_ref[i]
    o_ref[...] = jnp.where(tot * fb_ref[...] > 0, 1.0, -1.0).astype(jnp.float32)


def _tc_fbind(featp, wt, b2):
    return pl.pallas_call(
        _fb_body,
        out_shape=jax.ShapeDtypeStruct((1, _D), jnp.float32),
    )(featp, wt, b2)


def _tc_combine(partials, fbind):
    return pl.pallas_call(
        _comb_body,
        out_shape=jax.ShapeDtypeStruct((1, _D), jnp.float32),
    )(partials, fbind)


def kernel(input, feat, level_x, level_y, level_z, time_table, W, b):
    lows = jnp.array([0.0, -5.0, -5.0, -5.0], dtype=jnp.float32)
    spans = jnp.array([float(_TS), 10.0, 10.0, 10.0], dtype=jnp.float32)
    scales = jnp.array([_TS - 1.0, _LEVELS - 1.0, _LEVELS - 1.0, _LEVELS - 1.0],
                       dtype=jnp.float32)
    idx = jnp.round(
        jnp.clip((input - lows) / spans, 0.0, 1.0) * scales).astype(jnp.int32)
    idx = idx[:, jnp.array([1, 2, 3, 0])].reshape(_NW, _SPW, 4)

    partials = _sc_partials(idx, level_x, level_y, level_z, time_table)

    featp = feat.reshape(1, 6)
    wt = W.T
    b2 = b.reshape(1, _D)
    fbind = _tc_fbind(featp, wt, b2)
    out = _tc_combine(partials, fbind)
    return out.reshape(-1)
